# Initial kernel scaffold; baseline (speedup 1.0000x reference)
import jax
import jax.numpy as jnp
from jax.experimental import pallas as pl


def kernel(x, adj, gat1_fc_w, gat1_attn_w, gat2_fc_w, gat2_attn_w, sgc_w, sgc_b):
    raise NotImplementedError("write your pallas kernel here")


# speedup vs baseline: 2.4515x; 2.4515x over previous
import functools

import jax
import jax.numpy as jnp
from jax.experimental import pallas as pl
from jax.experimental.pallas import tpu as pltpu

_LAMBDA = 0.7
_PREC = jax.lax.Precision.HIGHEST
_BLOCKS = 8


def _elu(v):
    return jnp.where(v > 0, v, jnp.exp(v) - 1.0)


def _col_softmax(s):
    e = jnp.exp(s - jnp.max(s))
    return e / jnp.sum(e)


def _dot(a, b):
    return jnp.dot(a, b, precision=_PREC)


def _hop1_body(adj_ref, x_ref, fc1t_ref, aw1d_ref, hp_ref, y1_ref):
    x = x_ref[...]
    h = _dot(x, fc1t_ref[...])
    w1 = _col_softmax(_dot(h, aw1d_ref[...]))
    adj_b = adj_ref[...]
    hp_ref[...] = _elu(_dot(adj_b, w1 * h))
    y1_ref[...] = _dot(adj_b, x)


def _hop2_body(adj_ref, hp_ref, y1_ref, fc2t_ref, aw2d_ref, ns_ref, y2_ref):
    h2 = _dot(hp_ref[...], fc2t_ref[...])
    w2 = _col_softmax(h2 * aw2d_ref[0, 0])
    adj_b = adj_ref[...]
    ns_ref[...] = _elu(_dot(adj_b, w2 * h2))
    y2_ref[...] = _dot(adj_b, y1_ref[...])


def _hop3_body(adj_ref, y2_ref, ns_ref, y1b_ref, w1t_ref, w3t_ref, b_ref,
               out_ref):
    y3_b = _dot(adj_ref[...], y2_ref[...])
    keep = (ns_ref[...] > _LAMBDA).astype(jnp.float32)
    out_ref[...] = (_dot(keep * y1b_ref[...], w1t_ref[...])
                    + _dot((1.0 - keep) * y3_b, w3t_ref[...])
                    + b_ref[...])


def _row_block(rows, cols):
    return pl.BlockSpec((rows, cols), lambda i: (i, 0))


def _full(shape):
    return pl.BlockSpec(shape, lambda i: (0,) * len(shape))


def kernel(x, adj, gat1_fc_w, gat1_attn_w, gat2_fc_w, gat2_attn_w,
           sgc_w, sgc_b):
    n, f = x.shape
    hidden = gat1_fc_w.shape[0]
    out_f = sgc_w.shape[0]
    rb = n // _BLOCKS
    fc1t = gat1_fc_w.T
    aw1d = gat1_attn_w[:, hidden:].T
    fc2t = gat2_fc_w.T
    aw2d = gat2_attn_w[:, 1:2]
    w1t = sgc_w[:, :f].T
    w3t = sgc_w[:, 2 * f:].T
    b = sgc_b.reshape(1, out_f)

    call = functools.partial(
        pl.pallas_call,
        grid=(_BLOCKS,),
        compiler_params=pltpu.CompilerParams(
            dimension_semantics=("arbitrary",)),
    )

    hp, y1 = call(
        _hop1_body,
        in_specs=[_row_block(rb, n), _full((n, f)), _full((f, hidden)),
                  _full((hidden, 1))],
        out_specs=[_row_block(rb, hidden), _row_block(rb, f)],
        out_shape=[jax.ShapeDtypeStruct((n, hidden), jnp.float32),
                   jax.ShapeDtypeStruct((n, f), jnp.float32)],
    )(adj, x, fc1t, aw1d)

    ns, y2 = call(
        _hop2_body,
        in_specs=[_row_block(rb, n), _full((n, hidden)), _full((n, f)),
                  _full((hidden, 1)), _full((1, 1))],
        out_specs=[_row_block(rb, 1), _row_block(rb, f)],
        out_shape=[jax.ShapeDtypeStruct((n, 1), jnp.float32),
                   jax.ShapeDtypeStruct((n, f), jnp.float32)],
    )(adj, hp, y1, fc2t, aw2d)

    return call(
        _hop3_body,
        in_specs=[_row_block(rb, n), _full((n, f)), _row_block(rb, 1),
                  _row_block(rb, f), _full((f, out_f)), _full((f, out_f)),
                  _full((1, out_f))],
        out_specs=_row_block(rb, out_f),
        out_shape=jax.ShapeDtypeStruct((n, out_f), jnp.float32),
    )(adj, y2, ns, y1, w1t, w3t, b)


# speedup vs baseline: 6.1821x; 2.5217x over previous
import functools

import jax
import jax.numpy as jnp
from jax.experimental import pallas as pl
from jax.experimental.pallas import tpu as pltpu

_LAMBDA = 0.7
_PREC = jax.lax.Precision.DEFAULT
_BLOCKS = 8


def _elu(v):
    return jnp.where(v > 0, v, jnp.exp(v) - 1.0)


def _col_softmax(s):
    e = jnp.exp(s - jnp.max(s))
    return e / jnp.sum(e)


def _dot(a, b):
    return jnp.dot(a, b, precision=_PREC)


def _hop1_body(adj_ref, x_ref, fc1t_ref, aw1d_ref, hp_ref, y1_ref):
    x = x_ref[...]
    h = _dot(x, fc1t_ref[...])
    w1 = _col_softmax(_dot(h, aw1d_ref[...]))
    adj_b = adj_ref[...]
    hp_ref[...] = _elu(_dot(adj_b, w1 * h))
    y1_ref[...] = _dot(adj_b, x)


def _hop2_body(adj_ref, hp_ref, y1_ref, fc2t_ref, aw2d_ref, ns_ref, y2_ref):
    h2 = _dot(hp_ref[...], fc2t_ref[...])
    w2 = _col_softmax(h2 * aw2d_ref[0, 0])
    adj_b = adj_ref[...]
    ns_ref[...] = _elu(_dot(adj_b, w2 * h2))
    y2_ref[...] = _dot(adj_b, y1_ref[...])


def _hop3_body(adj_ref, y2_ref, ns_ref, y1b_ref, w1t_ref, w3t_ref, b_ref,
               out_ref):
    y3_b = _dot(adj_ref[...], y2_ref[...])
    keep = (ns_ref[...] > _LAMBDA).astype(jnp.float32)
    out_ref[...] = (_dot(keep * y1b_ref[...], w1t_ref[...])
                    + _dot((1.0 - keep) * y3_b, w3t_ref[...])
                    + b_ref[...])


def _row_block(rows, cols):
    return pl.BlockSpec((rows, cols), lambda i: (i, 0))


def _full(shape):
    return pl.BlockSpec(shape, lambda i: (0,) * len(shape))


def kernel(x, adj, gat1_fc_w, gat1_attn_w, gat2_fc_w, gat2_attn_w,
           sgc_w, sgc_b):
    n, f = x.shape
    hidden = gat1_fc_w.shape[0]
    out_f = sgc_w.shape[0]
    rb = n // _BLOCKS
    fc1t = gat1_fc_w.T
    aw1d = gat1_attn_w[:, hidden:].T
    fc2t = gat2_fc_w.T
    aw2d = gat2_attn_w[:, 1:2]
    w1t = sgc_w[:, :f].T
    w3t = sgc_w[:, 2 * f:].T
    b = sgc_b.reshape(1, out_f)

    call = functools.partial(
        pl.pallas_call,
        grid=(_BLOCKS,),
        compiler_params=pltpu.CompilerParams(
            dimension_semantics=("arbitrary",)),
    )

    hp, y1 = call(
        _hop1_body,
        in_specs=[_row_block(rb, n), _full((n, f)), _full((f, hidden)),
                  _full((hidden, 1))],
        out_specs=[_row_block(rb, hidden), _row_block(rb, f)],
        out_shape=[jax.ShapeDtypeStruct((n, hidden), jnp.float32),
                   jax.ShapeDtypeStruct((n, f), jnp.float32)],
    )(adj, x, fc1t, aw1d)

    ns, y2 = call(
        _hop2_body,
        in_specs=[_row_block(rb, n), _full((n, hidden)), _full((n, f)),
                  _full((hidden, 1)), _full((1, 1))],
        out_specs=[_row_block(rb, 1), _row_block(rb, f)],
        out_shape=[jax.ShapeDtypeStruct((n, 1), jnp.float32),
                   jax.ShapeDtypeStruct((n, f), jnp.float32)],
    )(adj, hp, y1, fc2t, aw2d)

    return call(
        _hop3_body,
        in_specs=[_row_block(rb, n), _full((n, f)), _row_block(rb, 1),
                  _row_block(rb, f), _full((f, out_f)), _full((f, out_f)),
                  _full((1, out_f))],
        out_specs=_row_block(rb, out_f),
        out_shape=jax.ShapeDtypeStruct((n, out_f), jnp.float32),
    )(adj, y2, ns, y1, w1t, w3t, b)


# speedup vs baseline: 8.1573x; 1.3195x over previous
import functools

import jax
import jax.numpy as jnp
from jax.experimental import pallas as pl
from jax.experimental.pallas import tpu as pltpu

_LAMBDA = 0.7
_BLOCKS = 4


def _elu(v):
    return jnp.where(v > 0, v, jnp.exp(v) - 1.0)


def _col_softmax(s):
    e = jnp.exp(s - jnp.max(s))
    return e / jnp.sum(e)


def _hop1_body(adj_ref, x_ref, fc1t_ref, aw1d_ref, hp_ref, y1_ref, g1_ref):
    @pl.when(pl.program_id(0) == 0)
    def _prep():
        h = jnp.dot(x_ref[...], fc1t_ref[...])
        w1 = _col_softmax(jnp.dot(h, aw1d_ref[...]))
        g1_ref[...] = w1 * h

    adj_b = adj_ref[...]
    hp_ref[...] = _elu(jnp.dot(adj_b, g1_ref[...]))
    y1_ref[...] = jnp.dot(adj_b, x_ref[...])


def _hop2_body(adj_ref, hp_ref, y1_ref, fc2t_ref, aw2d_ref, ns_ref, y2_ref,
               g2_ref):
    @pl.when(pl.program_id(0) == 0)
    def _prep():
        h2 = jnp.dot(hp_ref[...], fc2t_ref[...])
        g2_ref[...] = _col_softmax(h2 * aw2d_ref[0, 0]) * h2

    adj_b = adj_ref[...]
    ns_ref[...] = _elu(jnp.dot(adj_b, g2_ref[...]))
    y2_ref[...] = jnp.dot(adj_b, y1_ref[...])


def _hop3_body(adj_ref, y2_ref, ns_ref, y1b_ref, w1t_ref, w3t_ref, b_ref,
               out_ref):
    y3_b = jnp.dot(adj_ref[...], y2_ref[...])
    keep = (ns_ref[...] > _LAMBDA).astype(jnp.float32)
    out_ref[...] = (jnp.dot(keep * y1b_ref[...], w1t_ref[...])
                    + jnp.dot((1.0 - keep) * y3_b, w3t_ref[...])
                    + b_ref[...])


def _row_block(rows, cols):
    return pl.BlockSpec((rows, cols), lambda i: (i, 0))


def _full(shape):
    return pl.BlockSpec(shape, lambda i: (0,) * len(shape))


def kernel(x, adj, gat1_fc_w, gat1_attn_w, gat2_fc_w, gat2_attn_w,
           sgc_w, sgc_b):
    n, f = x.shape
    hidden = gat1_fc_w.shape[0]
    out_f = sgc_w.shape[0]
    rb = n // _BLOCKS
    fc1t = gat1_fc_w.T
    aw1d = gat1_attn_w[:, hidden:].T
    fc2t = gat2_fc_w.T
    aw2d = gat2_attn_w[:, 1:2]
    w1t = sgc_w[:, :f].T
    w3t = sgc_w[:, 2 * f:].T
    b = sgc_b.reshape(1, out_f)

    call = functools.partial(
        pl.pallas_call,
        grid=(_BLOCKS,),
        compiler_params=pltpu.CompilerParams(
            dimension_semantics=("arbitrary",)),
    )

    hp, y1 = call(
        _hop1_body,
        in_specs=[_row_block(rb, n), _full((n, f)), _full((f, hidden)),
                  _full((hidden, 1))],
        out_specs=[_row_block(rb, hidden), _row_block(rb, f)],
        out_shape=[jax.ShapeDtypeStruct((n, hidden), jnp.float32),
                   jax.ShapeDtypeStruct((n, f), jnp.float32)],
        scratch_shapes=[pltpu.VMEM((n, hidden), jnp.float32)],
    )(adj, x, fc1t, aw1d)

    ns, y2 = call(
        _hop2_body,
        in_specs=[_row_block(rb, n), _full((n, hidden)), _full((n, f)),
                  _full((hidden, 1)), _full((1, 1))],
        out_specs=[_row_block(rb, 1), _row_block(rb, f)],
        out_shape=[jax.ShapeDtypeStruct((n, 1), jnp.float32),
                   jax.ShapeDtypeStruct((n, f), jnp.float32)],
        scratch_shapes=[pltpu.VMEM((n, 1), jnp.float32)],
    )(adj, hp, y1, fc2t, aw2d)

    return call(
        _hop3_body,
        in_specs=[_row_block(rb, n), _full((n, f)), _row_block(rb, 1),
                  _row_block(rb, f), _full((f, out_f)), _full((f, out_f)),
                  _full((1, out_f))],
        out_specs=_row_block(rb, out_f),
        out_shape=jax.ShapeDtypeStruct((n, out_f), jnp.float32),
    )(adj, y2, ns, y1, w1t, w3t, b)


# speedup vs baseline: 8.2481x; 1.0111x over previous
import functools

import jax
import jax.numpy as jnp
from jax.experimental import pallas as pl
from jax.experimental.pallas import tpu as pltpu

_LAMBDA = 0.7
_BLOCKS = 2


def _elu(v):
    return jnp.where(v > 0, v, jnp.exp(v) - 1.0)


def _col_softmax(s):
    e = jnp.exp(s - jnp.max(s))
    return e / jnp.sum(e)


def _hop1_body(adj_ref, x_ref, fc1t_ref, aw1d_ref, hp_ref, y1_ref, g1_ref):
    @pl.when(pl.program_id(0) == 0)
    def _prep():
        h = jnp.dot(x_ref[...], fc1t_ref[...])
        w1 = _col_softmax(jnp.dot(h, aw1d_ref[...]))
        g1_ref[...] = w1 * h

    adj_b = adj_ref[...]
    hp_ref[...] = _elu(jnp.dot(adj_b, g1_ref[...]))
    y1_ref[...] = jnp.dot(adj_b, x_ref[...])


def _hop2_body(adj_ref, hp_ref, y1_ref, fc2t_ref, aw2d_ref, ns_ref, y2_ref,
               g2_ref):
    @pl.when(pl.program_id(0) == 0)
    def _prep():
        h2 = jnp.dot(hp_ref[...], fc2t_ref[...])
        g2_ref[...] = _col_softmax(h2 * aw2d_ref[0, 0]) * h2

    adj_b = adj_ref[...]
    ns_ref[...] = _elu(jnp.dot(adj_b, g2_ref[...]))
    y2_ref[...] = jnp.dot(adj_b, y1_ref[...])


def _hop3_body(adj_ref, y2_ref, ns_ref, y1b_ref, w1t_ref, w3t_ref, b_ref,
               out_ref):
    y3_b = jnp.dot(adj_ref[...], y2_ref[...])
    keep = (ns_ref[...] > _LAMBDA).astype(jnp.float32)
    out_ref[...] = (jnp.dot(keep * y1b_ref[...], w1t_ref[...])
                    + jnp.dot((1.0 - keep) * y3_b, w3t_ref[...])
                    + b_ref[...])


def _row_block(rows, cols):
    return pl.BlockSpec((rows, cols), lambda i: (i, 0))


def _full(shape):
    return pl.BlockSpec(shape, lambda i: (0,) * len(shape))


def kernel(x, adj, gat1_fc_w, gat1_attn_w, gat2_fc_w, gat2_attn_w,
           sgc_w, sgc_b):
    n, f = x.shape
    hidden = gat1_fc_w.shape[0]
    out_f = sgc_w.shape[0]
    rb = n // _BLOCKS
    fc1t = gat1_fc_w.T
    aw1d = gat1_attn_w[:, hidden:].T
    fc2t = gat2_fc_w.T
    aw2d = gat2_attn_w[:, 1:2]
    w1t = sgc_w[:, :f].T
    w3t = sgc_w[:, 2 * f:].T
    b = sgc_b.reshape(1, out_f)

    call = functools.partial(
        pl.pallas_call,
        grid=(_BLOCKS,),
        compiler_params=pltpu.CompilerParams(
            dimension_semantics=("arbitrary",)),
    )

    hp, y1 = call(
        _hop1_body,
        in_specs=[_row_block(rb, n), _full((n, f)), _full((f, hidden)),
                  _full((hidden, 1))],
        out_specs=[_row_block(rb, hidden), _row_block(rb, f)],
        out_shape=[jax.ShapeDtypeStruct((n, hidden), jnp.float32),
                   jax.ShapeDtypeStruct((n, f), jnp.float32)],
        scratch_shapes=[pltpu.VMEM((n, hidden), jnp.float32)],
    )(adj, x, fc1t, aw1d)

    ns, y2 = call(
        _hop2_body,
        in_specs=[_row_block(rb, n), _full((n, hidden)), _full((n, f)),
                  _full((hidden, 1)), _full((1, 1))],
        out_specs=[_row_block(rb, 1), _row_block(rb, f)],
        out_shape=[jax.ShapeDtypeStruct((n, 1), jnp.float32),
                   jax.ShapeDtypeStruct((n, f), jnp.float32)],
        scratch_shapes=[pltpu.VMEM((n, 1), jnp.float32)],
    )(adj, hp, y1, fc2t, aw2d)

    return call(
        _hop3_body,
        in_specs=[_row_block(rb, n), _full((n, f)), _row_block(rb, 1),
                  _row_block(rb, f), _full((f, out_f)), _full((f, out_f)),
                  _full((1, out_f))],
        out_specs=_row_block(rb, out_f),
        out_shape=jax.ShapeDtypeStruct((n, out_f), jnp.float32),
    )(adj, y2, ns, y1, w1t, w3t, b)


# speedup vs baseline: 10.1105x; 1.2258x over previous
import jax
import jax.numpy as jnp
from jax.experimental import pallas as pl
from jax.experimental.pallas import tpu as pltpu

_LAMBDA = 0.7
_CHUNKS = 4


def _elu(v):
    return jnp.where(v > 0, v, jnp.exp(v) - 1.0)


def _col_softmax(s):
    e = jnp.exp(s - jnp.max(s))
    return e / jnp.sum(e)


def _fused_body(adj_ref, x_ref, fc1t_ref, aw1d_ref, fc2t_ref, aw2d_ref,
                w1t_ref, w3t_ref, b_ref, out_ref,
                y1_ref, y2_ref, hp_ref, ns_ref):
    n = adj_ref.shape[0]
    rb = n // _CHUNKS
    x = x_ref[...]

    h = jnp.dot(x, fc1t_ref[...])
    g1 = _col_softmax(jnp.dot(h, aw1d_ref[...])) * h

    for c in range(_CHUNKS):
        rows = pl.ds(c * rb, rb)
        adj_b = adj_ref[rows, :]
        hp_ref[rows, :] = _elu(jnp.dot(adj_b, g1))
        y1_ref[rows, :] = jnp.dot(adj_b, x)

    h2 = jnp.dot(hp_ref[...], fc2t_ref[...])
    g2 = _col_softmax(h2 * aw2d_ref[0, 0]) * h2

    for c in range(_CHUNKS):
        rows = pl.ds(c * rb, rb)
        adj_b = adj_ref[rows, :]
        ns_ref[rows, :] = _elu(jnp.dot(adj_b, g2))
        y2_ref[rows, :] = jnp.dot(adj_b, y1_ref[...])

    for c in range(_CHUNKS):
        rows = pl.ds(c * rb, rb)
        y3_b = jnp.dot(adj_ref[rows, :], y2_ref[...])
        keep = (ns_ref[rows, :] > _LAMBDA).astype(jnp.float32)
        out_ref[rows, :] = (jnp.dot(keep * y1_ref[rows, :], w1t_ref[...])
                            + jnp.dot((1.0 - keep) * y3_b, w3t_ref[...])
                            + b_ref[...])


def kernel(x, adj, gat1_fc_w, gat1_attn_w, gat2_fc_w, gat2_attn_w,
           sgc_w, sgc_b):
    n, f = x.shape
    hidden = gat1_fc_w.shape[0]
    out_f = sgc_w.shape[0]
    fc1t = gat1_fc_w.T
    aw1d = gat1_attn_w[:, hidden:].T
    fc2t = gat2_fc_w.T
    aw2d = gat2_attn_w[:, 1:2]
    w1t = sgc_w[:, :f].T
    w3t = sgc_w[:, 2 * f:].T
    b = sgc_b.reshape(1, out_f)

    return pl.pallas_call(
        _fused_body,
        out_shape=jax.ShapeDtypeStruct((n, out_f), jnp.float32),
        scratch_shapes=[
            pltpu.VMEM((n, f), jnp.float32),
            pltpu.VMEM((n, f), jnp.float32),
            pltpu.VMEM((n, hidden), jnp.float32),
            pltpu.VMEM((n, 1), jnp.float32),
        ],
        compiler_params=pltpu.CompilerParams(
            vmem_limit_bytes=60 * 1024 * 1024),
    )(adj, x, fc1t, aw1d, fc2t, aw2d, w1t, w3t, b)


# speedup vs baseline: 10.8047x; 1.0687x over previous
import jax
import jax.numpy as jnp
from jax.experimental import pallas as pl
from jax.experimental.pallas import tpu as pltpu

_LAMBDA = 0.7
_CHUNKS = 8


def _elu(v):
    return jnp.where(v > 0, v, jnp.exp(v) - 1.0)


def _col_softmax(s):
    e = jnp.exp(s - jnp.max(s))
    return e / jnp.sum(e)


def _fused_body(adj_hbm, x_ref, fc1t_ref, aw1d_ref, fc2t_ref, aw2d_ref,
                w1t_ref, w3t_ref, b_ref, out_ref,
                adj_ref, y1_ref, y2_ref, hp_ref, ns_ref, sems):
    n = out_ref.shape[0]
    rb = n // _CHUNKS

    def _chunk_copy(c):
        rows = pl.ds(c * rb, rb)
        return pltpu.make_async_copy(adj_hbm.at[rows, :],
                                     adj_ref.at[rows, :], sems.at[c])

    for c in range(_CHUNKS):
        _chunk_copy(c).start()

    x = x_ref[...]

    h = jnp.dot(x, fc1t_ref[...])
    g1 = _col_softmax(jnp.dot(h, aw1d_ref[...])) * h

    for c in range(_CHUNKS):
        _chunk_copy(c).wait()
        rows = pl.ds(c * rb, rb)
        adj_b = adj_ref[rows, :]
        hp_ref[rows, :] = _elu(jnp.dot(adj_b, g1))
        y1_ref[rows, :] = jnp.dot(adj_b, x)

    h2 = jnp.dot(hp_ref[...], fc2t_ref[...])
    g2 = _col_softmax(h2 * aw2d_ref[0, 0]) * h2

    for c in range(_CHUNKS):
        rows = pl.ds(c * rb, rb)
        adj_b = adj_ref[rows, :]
        ns_ref[rows, :] = _elu(jnp.dot(adj_b, g2))
        y2_ref[rows, :] = jnp.dot(adj_b, y1_ref[...])

    for c in range(_CHUNKS):
        rows = pl.ds(c * rb, rb)
        y3_b = jnp.dot(adj_ref[rows, :], y2_ref[...])
        keep = (ns_ref[rows, :] > _LAMBDA).astype(jnp.float32)
        out_ref[rows, :] = (jnp.dot(keep * y1_ref[rows, :], w1t_ref[...])
                            + jnp.dot((1.0 - keep) * y3_b, w3t_ref[...])
                            + b_ref[...])


def kernel(x, adj, gat1_fc_w, gat1_attn_w, gat2_fc_w, gat2_attn_w,
           sgc_w, sgc_b):
    n, f = x.shape
    hidden = gat1_fc_w.shape[0]
    out_f = sgc_w.shape[0]
    fc1t = gat1_fc_w.T
    aw1d = gat1_attn_w[:, hidden:].T
    fc2t = gat2_fc_w.T
    aw2d = gat2_attn_w[:, 1:2]
    w1t = sgc_w[:, :f].T
    w3t = sgc_w[:, 2 * f:].T
    b = sgc_b.reshape(1, out_f)

    vmem = pl.BlockSpec(memory_space=pltpu.MemorySpace.VMEM)
    return pl.pallas_call(
        _fused_body,
        in_specs=[pl.BlockSpec(memory_space=pltpu.MemorySpace.HBM)]
        + [vmem] * 8,
        out_specs=vmem,
        out_shape=jax.ShapeDtypeStruct((n, out_f), jnp.float32),
        scratch_shapes=[
            pltpu.VMEM((n, n), jnp.float32),
            pltpu.VMEM((n, f), jnp.float32),
            pltpu.VMEM((n, f), jnp.float32),
            pltpu.VMEM((n, hidden), jnp.float32),
            pltpu.VMEM((n, 1), jnp.float32),
            pltpu.SemaphoreType.DMA((_CHUNKS,)),
        ],
        compiler_params=pltpu.CompilerParams(
            vmem_limit_bytes=60 * 1024 * 1024),
    )(adj, x, fc1t, aw1d, fc2t, aw2d, w1t, w3t, b)


# speedup vs baseline: 11.7774x; 1.0900x over previous
import jax
import jax.numpy as jnp
from jax.experimental import pallas as pl
from jax.experimental.pallas import tpu as pltpu

_LAMBDA = 0.7
_CHUNKS = 8


def _elu(v):
    return jnp.where(v > 0, v, jnp.exp(v) - 1.0)


def _col_softmax(s):
    e = jnp.exp(s - jnp.max(s))
    return e / jnp.sum(e)


def _fused_body(adj_hbm, x_ref, fc1t_ref, aw1d_ref, fc2t_ref, aw2d_ref,
                w1t_ref, w3t_ref, b_ref, out_ref,
                adj_ref, y1_ref, y2_ref, hp_ref, ns_ref, rhs_ref, sems):
    n = out_ref.shape[0]
    rb = n // _CHUNKS

    def _chunk_copy(c):
        rows = pl.ds(c * rb, rb)
        return pltpu.make_async_copy(adj_hbm.at[rows, :],
                                     adj_ref.at[rows, :], sems.at[c])

    for c in range(_CHUNKS):
        _chunk_copy(c).start()

    x = x_ref[...]
    f = x.shape[1]

    h = jnp.dot(x, fc1t_ref[...])
    g1 = _col_softmax(jnp.dot(h, aw1d_ref[...])) * h
    rhs_ref[:, :f] = x
    rhs_ref[:, f:] = jnp.pad(g1, ((0, 0), (0, f - g1.shape[1])))

    for c in range(_CHUNKS):
        _chunk_copy(c).wait()
        rows = pl.ds(c * rb, rb)
        m1 = jnp.dot(adj_ref[rows, :], rhs_ref[...])
        y1_ref[rows, :] = m1[:, :f]
        hp_ref[rows, :] = _elu(m1[:, f:f + hp_ref.shape[1]])

    h2 = jnp.dot(hp_ref[...], fc2t_ref[...])
    g2 = _col_softmax(h2 * aw2d_ref[0, 0]) * h2
    rhs_ref[:, :f] = y1_ref[...]
    rhs_ref[:, f:] = jnp.pad(g2, ((0, 0), (0, f - g2.shape[1])))

    for c in range(_CHUNKS):
        rows = pl.ds(c * rb, rb)
        m2 = jnp.dot(adj_ref[rows, :], rhs_ref[...])
        y2_ref[rows, :] = m2[:, :f]
        ns_ref[rows, :] = _elu(m2[:, f:f + 1])

    for c in range(_CHUNKS):
        rows = pl.ds(c * rb, rb)
        y3_b = jnp.dot(adj_ref[rows, :], y2_ref[...])
        keep = (ns_ref[rows, :] > _LAMBDA).astype(jnp.float32)
        out_ref[rows, :] = (jnp.dot(keep * y1_ref[rows, :], w1t_ref[...])
                            + jnp.dot((1.0 - keep) * y3_b, w3t_ref[...])
                            + b_ref[...])


def kernel(x, adj, gat1_fc_w, gat1_attn_w, gat2_fc_w, gat2_attn_w,
           sgc_w, sgc_b):
    n, f = x.shape
    hidden = gat1_fc_w.shape[0]
    out_f = sgc_w.shape[0]
    fc1t = gat1_fc_w.T
    aw1d = gat1_attn_w[:, hidden:].T
    fc2t = gat2_fc_w.T
    aw2d = gat2_attn_w[:, 1:2]
    w1t = sgc_w[:, :f].T
    w3t = sgc_w[:, 2 * f:].T
    b = sgc_b.reshape(1, out_f)

    vmem = pl.BlockSpec(memory_space=pltpu.MemorySpace.VMEM)
    return pl.pallas_call(
        _fused_body,
        in_specs=[pl.BlockSpec(memory_space=pltpu.MemorySpace.HBM)]
        + [vmem] * 8,
        out_specs=vmem,
        out_shape=jax.ShapeDtypeStruct((n, out_f), jnp.float32),
        scratch_shapes=[
            pltpu.VMEM((n, n), jnp.float32),
            pltpu.VMEM((n, f), jnp.float32),
            pltpu.VMEM((n, f), jnp.float32),
            pltpu.VMEM((n, hidden), jnp.float32),
            pltpu.VMEM((n, 1), jnp.float32),
            pltpu.VMEM((n, 2 * f), jnp.float32),
            pltpu.SemaphoreType.DMA((_CHUNKS,)),
        ],
        compiler_params=pltpu.CompilerParams(
            vmem_limit_bytes=60 * 1024 * 1024),
    )(adj, x, fc1t, aw1d, fc2t, aw2d, w1t, w3t, b)


# speedup vs baseline: 16.3960x; 1.3922x over previous
import jax
import jax.numpy as jnp
from jax.experimental import pallas as pl
from jax.experimental.pallas import tpu as pltpu

_LAMBDA = 0.7
_CHUNKS = 8


def _elu(v):
    return jnp.where(v > 0, v, jnp.exp(v) - 1.0)


def _col_softmax(s):
    e = jnp.exp(s - jnp.max(s))
    return e / jnp.sum(e)


def _dot_t(a, b):
    return jax.lax.dot_general(a, b, (((1,), (1,)), ((), ())))


def _fused_body(adj_hbm, x_ref, fc1_ref, aw1_ref, fc2_ref, aw2_ref,
                sgcw_ref, b_ref, out_ref,
                adj_ref, y1_ref, y2_ref, hp_ref, ns_ref, rhs_ref, sems):
    n = adj_ref.shape[0]
    rb = n // _CHUNKS

    def _chunk_copy(c):
        rows = pl.ds(c * rb, rb)
        return pltpu.make_async_copy(adj_hbm.at[rows, :],
                                     adj_ref.at[rows, :], sems.at[c])

    for c in range(_CHUNKS):
        _chunk_copy(c).start()

    x = x_ref[...]
    f = x.shape[1]
    hidden = fc1_ref.shape[0]

    h = _dot_t(x, fc1_ref[...])
    g1 = _col_softmax(_dot_t(h, aw1_ref[:, hidden:])) * h
    rhs_ref[:, :f] = x
    rhs_ref[:, f:] = jnp.pad(g1, ((0, 0), (0, f - hidden)))

    for c in range(_CHUNKS):
        _chunk_copy(c).wait()
        rows = pl.ds(c * rb, rb)
        m1 = jnp.dot(adj_ref[rows, :], rhs_ref[...])
        y1_ref[rows, :] = m1[:, :f]
        hp_ref[rows, :] = _elu(m1[:, f:f + hidden])

    h2 = _dot_t(hp_ref[...], fc2_ref[...])
    g2 = _col_softmax(h2 * aw2_ref[0, 1]) * h2
    rhs_ref[:, :f] = y1_ref[...]
    rhs_ref[:, f:] = jnp.pad(g2, ((0, 0), (0, f - 1)))

    for c in range(_CHUNKS):
        rows = pl.ds(c * rb, rb)
        m2 = jnp.dot(adj_ref[rows, :], rhs_ref[...])
        y2_ref[rows, :] = m2[:, :f]
        ns_ref[rows, :] = _elu(m2[:, f:f + 1])

    for c in range(_CHUNKS):
        rows = pl.ds(c * rb, rb)
        y3_b = jnp.dot(adj_ref[rows, :], y2_ref[...])
        keep = (ns_ref[rows, :] > _LAMBDA).astype(jnp.float32)
        out_ref[rows, :] = (_dot_t(keep * y1_ref[rows, :], sgcw_ref[:, :f])
                            + _dot_t((1.0 - keep) * y3_b,
                                     sgcw_ref[:, 2 * f:])
                            + b_ref[...])


def kernel(x, adj, gat1_fc_w, gat1_attn_w, gat2_fc_w, gat2_attn_w,
           sgc_w, sgc_b):
    n, f = x.shape
    hidden = gat1_fc_w.shape[0]
    out_f = sgc_w.shape[0]

    vmem = pl.BlockSpec(memory_space=pltpu.MemorySpace.VMEM)
    return pl.pallas_call(
        _fused_body,
        in_specs=[pl.BlockSpec(memory_space=pltpu.MemorySpace.HBM)]
        + [vmem] * 7,
        out_specs=vmem,
        out_shape=jax.ShapeDtypeStruct((n, out_f), jnp.float32),
        scratch_shapes=[
            pltpu.VMEM((n, n), jnp.float32),
            pltpu.VMEM((n, f), jnp.float32),
            pltpu.VMEM((n, f), jnp.float32),
            pltpu.VMEM((n, hidden), jnp.float32),
            pltpu.VMEM((n, 1), jnp.float32),
            pltpu.VMEM((n, 2 * f), jnp.float32),
            pltpu.SemaphoreType.DMA((_CHUNKS,)),
        ],
        compiler_params=pltpu.CompilerParams(
            vmem_limit_bytes=60 * 1024 * 1024),
    )(adj, x, gat1_fc_w, gat1_attn_w, gat2_fc_w, gat2_attn_w, sgc_w,
      sgc_b.reshape(1, out_f))


# speedup vs baseline: 18.8703x; 1.1509x over previous
import jax
import jax.numpy as jnp
from jax.experimental import pallas as pl
from jax.experimental.pallas import tpu as pltpu

_LAMBDA = 0.7
_CHUNKS = 8


def _elu(v):
    return jnp.where(v > 0, v, jnp.exp(v) - 1.0)


def _col_softmax(s):
    e = jnp.exp(s - jnp.max(s))
    return e / jnp.sum(e)


def _dot_t(a, b):
    return jax.lax.dot_general(a, b, (((1,), (1,)), ((), ())))


def _fused_body(adj_hbm, x_ref, fc1_ref, aw1_ref, fc2_ref, aw2_ref,
                sgcw_ref, b_ref, out_ref,
                adj_ref, rhs_ref, t1_ref, v_ref, hp_ref, ns_ref, sems):
    n = adj_ref.shape[0]
    rb = n // _CHUNKS

    def _chunk_copy(c):
        rows = pl.ds(c * rb, rb)
        return pltpu.make_async_copy(adj_hbm.at[rows, :],
                                     adj_ref.at[rows, :], sems.at[c])

    for c in range(_CHUNKS):
        _chunk_copy(c).start()

    x = x_ref[...]
    f = x.shape[1]
    hidden = fc1_ref.shape[0]
    out_f = b_ref.shape[1]

    h = _dot_t(x, fc1_ref[...])
    g1 = _col_softmax(_dot_t(h, aw1_ref[:, hidden:])) * h
    u1 = _dot_t(x, sgcw_ref[:, :f])
    u3 = _dot_t(x, sgcw_ref[:, 2 * f:])
    rhs_ref[...] = jnp.concatenate([u1, u3, g1], axis=1)

    c1 = 2 * out_f
    for c in range(_CHUNKS):
        _chunk_copy(c).wait()
        rows = pl.ds(c * rb, rb)
        m1 = jnp.dot(adj_ref[rows, :], rhs_ref[...])
        t1_ref[rows, :] = m1[:, :out_f]
        v_ref[rows, :out_f] = m1[:, out_f:c1]
        hp_ref[rows, :] = _elu(m1[:, c1:c1 + hidden])

    h2 = _dot_t(hp_ref[...], fc2_ref[...])
    g2 = _col_softmax(h2 * aw2_ref[0, 1]) * h2
    v_ref[:, out_f:] = g2

    for c in range(_CHUNKS):
        rows = pl.ds(c * rb, rb)
        m2 = jnp.dot(adj_ref[rows, :], v_ref[...])
        ns_ref[rows, :] = _elu(m2[:, out_f:])
        v_ref2 = m2[:, :out_f]
        rhs_ref[rows, :out_f] = v_ref2

    for c in range(_CHUNKS):
        rows = pl.ds(c * rb, rb)
        m3 = jnp.dot(adj_ref[rows, :], rhs_ref[:, :out_f])
        keep = (ns_ref[rows, :] > _LAMBDA).astype(jnp.float32)
        out_ref[rows, :] = (keep * t1_ref[rows, :] + (1.0 - keep) * m3
                            + b_ref[...])


def kernel(x, adj, gat1_fc_w, gat1_attn_w, gat2_fc_w, gat2_attn_w,
           sgc_w, sgc_b):
    n, f = x.shape
    hidden = gat1_fc_w.shape[0]
    out_f = sgc_w.shape[0]

    vmem = pl.BlockSpec(memory_space=pltpu.MemorySpace.VMEM)
    return pl.pallas_call(
        _fused_body,
        in_specs=[pl.BlockSpec(memory_space=pltpu.MemorySpace.HBM)]
        + [vmem] * 7,
        out_specs=vmem,
        out_shape=jax.ShapeDtypeStruct((n, out_f), jnp.float32),
        scratch_shapes=[
            pltpu.VMEM((n, n), jnp.float32),
            pltpu.VMEM((n, 2 * out_f + hidden), jnp.float32),
            pltpu.VMEM((n, out_f), jnp.float32),
            pltpu.VMEM((n, out_f + 1), jnp.float32),
            pltpu.VMEM((n, hidden), jnp.float32),
            pltpu.VMEM((n, 1), jnp.float32),
            pltpu.SemaphoreType.DMA((_CHUNKS,)),
        ],
        compiler_params=pltpu.CompilerParams(
            vmem_limit_bytes=60 * 1024 * 1024),
    )(adj, x, gat1_fc_w, gat1_attn_w, gat2_fc_w, gat2_attn_w, sgc_w,
      sgc_b.reshape(1, out_f))
